# Initial kernel scaffold; baseline (speedup 1.0000x reference)
import jax
import jax.numpy as jnp
from jax.experimental import pallas as pl


def kernel(x, boundary, edge_index, relation_weight, W, b, gamma, beta):
    raise NotImplementedError("write your pallas kernel here")


# speedup vs baseline: 4.2216x; 4.2216x over previous
import functools

import jax
import jax.numpy as jnp
from jax import lax
from jax.experimental import pallas as pl
from jax.experimental.pallas import tpu as pltpu
from jax.experimental.pallas import tpu_sc as plsc

_N = 10000
_E = 320000
_D = 128

_NC = 2
_NS = 16
_CH = 128
_CHUNKS_PER_TILE = 79
_T = _CH * _CHUNKS_PER_TILE
_E_PAD = _NC * _NS * _T
_N_PAD = 10112
_RPT = _N_PAD // _NS


def _sc_segment_partials(x, src_p, dst_p, zeros):
  mesh = plsc.VectorSubcoreMesh(
      core_axis_name="c", subcore_axis_name="s",
      num_cores=_NC, num_subcores=_NS)

  @functools.partial(
      pl.kernel,
      out_type=jax.ShapeDtypeStruct((_NC, _N_PAD, _D), jnp.float32),
      mesh=mesh,
      scratch_types=[
          pltpu.VMEM((_CH,), jnp.int32),
          pltpu.VMEM((_CH,), jnp.int32),
          pltpu.VMEM((_CH, _D), jnp.float32),
          pltpu.VMEM_SHARED((_N_PAD, _D), jnp.float32),
          pltpu.SemaphoreType.DMA,
      ],
  )
  def sc_kernel(x_hbm, src_hbm, dst_hbm, z_hbm, out_hbm,
                sidx, didx, rows, agg, sem):
    c = lax.axis_index("c")
    s = lax.axis_index("s")
    pltpu.sync_copy(z_hbm.at[pl.ds(s * _RPT, _RPT)],
                    agg.at[pl.ds(s * _RPT, _RPT)])
    plsc.subcore_barrier()

    tile_base = c * (_E_PAD // _NC) + s * _T

    def chunk_body(i, carry):
      base = tile_base + i * _CH
      pltpu.sync_copy(src_hbm.at[pl.ds(base, _CH)], sidx)
      pltpu.sync_copy(dst_hbm.at[pl.ds(base, _CH)], didx)
      pltpu.async_copy(x_hbm.at[sidx], rows, sem).wait()
      pltpu.sync_copy(rows, agg.at[didx], add=True)
      return carry

    lax.fori_loop(0, _CHUNKS_PER_TILE, chunk_body, 0)
    plsc.subcore_barrier()
    pltpu.sync_copy(agg.at[pl.ds(s * _RPT, _RPT)],
                    out_hbm.at[c, pl.ds(s * _RPT, _RPT)])

  return sc_kernel(x, src_p, dst_p, zeros)


_BLK = 400


def _tc_epilogue(x, partial, boundary, rw, W, b2, g2, be2):
  def body(x_ref, p_ref, bnd_ref, rw_ref, w_ref, b_ref, g_ref, be_ref, o_ref):
    agg = (p_ref[0] + p_ref[1]) * rw_ref[...] + bnd_ref[...]
    hcat = jnp.concatenate([x_ref[...], agg], axis=-1)
    h = lax.dot_general(
        hcat, w_ref[...], (((1,), (1,)), ((), ())),
        preferred_element_type=jnp.float32,
        precision=lax.Precision.HIGHEST) + b_ref[...]
    mean = jnp.mean(h, axis=-1, keepdims=True)
    hc = h - mean
    var = jnp.mean(hc * hc, axis=-1, keepdims=True)
    h = hc * lax.rsqrt(var + 1e-5) * g_ref[...] + be_ref[...]
    o_ref[...] = jnp.maximum(h, 0.0)

  grid = (_N // _BLK,)
  return pl.pallas_call(
      body,
      grid=grid,
      in_specs=[
          pl.BlockSpec((_BLK, _D), lambda i: (i, 0)),
          pl.BlockSpec((_NC, _BLK, _D), lambda i: (0, i, 0)),
          pl.BlockSpec((_BLK, _D), lambda i: (i, 0)),
          pl.BlockSpec((1, _D), lambda i: (0, 0)),
          pl.BlockSpec((_D, 2 * _D), lambda i: (0, 0)),
          pl.BlockSpec((1, _D), lambda i: (0, 0)),
          pl.BlockSpec((1, _D), lambda i: (0, 0)),
          pl.BlockSpec((1, _D), lambda i: (0, 0)),
      ],
      out_specs=pl.BlockSpec((_BLK, _D), lambda i: (i, 0)),
      out_shape=jax.ShapeDtypeStruct((_N, _D), jnp.float32),
  )(x, partial, boundary, rw, W, b2, g2, be2)


def kernel(x, boundary, edge_index, relation_weight, W, b, gamma, beta):
  pad = _E_PAD - _E
  src_p = jnp.concatenate([edge_index[0], jnp.zeros((pad,), jnp.int32)])
  dst_p = jnp.concatenate(
      [edge_index[1], jnp.full((pad,), _N_PAD - 1, jnp.int32)])
  zeros = jnp.zeros((_N_PAD, _D), jnp.float32)

  partial = _sc_segment_partials(x, src_p, dst_p, zeros)

  return _tc_epilogue(
      x, partial, boundary,
      relation_weight.reshape(1, _D), W,
      b.reshape(1, _D), gamma.reshape(1, _D), beta.reshape(1, _D))
